# baseline (speedup 1.0000x reference)
import functools

import jax
import jax.numpy as jnp
from jax import lax
from jax.experimental import pallas as pl
from jax.experimental.pallas import tpu as pltpu
from jax.experimental.pallas import tpu_sc as plsc

B = 16384
N = 256
L = 16
NCHUNK = N // L
PAD = L + 1

_info = plsc.get_sparse_core_info()
NC, NS = _info.num_cores, _info.num_subcores
NW = NC * NS
ROWS_W = B // NW
CH = 64
NCH = ROWS_W // CH

_BIG = 2**30


def _wtac_body(x_hbm, lab_hbm, out_hbm, xb0, xb1, labv, outv, tbuf, ibuf,
               sem0, sem1, osem):
    wid = lax.axis_index("s") * NC + lax.axis_index("c")
    base = wid * ROWS_W * N
    pltpu.sync_copy(lab_hbm, labv)
    bufs = (xb0, xb1)
    sems = (sem0, sem1)
    copies = [None] * NCH
    copies[0] = pltpu.async_copy(
        x_hbm.at[pl.ds(base, CH * N)], xb0, sems[0]
    )
    lane = lax.iota(jnp.int32, L)
    idxs = [lane + jnp.int32(k * L) for k in range(NCHUNK)]
    lane_pad = lane * jnp.int32(PAD)

    for ch in range(NCH):
        if ch + 1 < NCH:
            copies[ch + 1] = pltpu.async_copy(
                x_hbm.at[pl.ds(base + (ch + 1) * CH * N, CH * N)],
                bufs[(ch + 1) % 2],
                sems[(ch + 1) % 2],
            )
        copies[ch].wait()
        xch = bufs[ch % 2]

        def group_step(g, carry, xch=xch):
            go = g * (L * N)
            for rr in range(L):
                off = go + rr * N
                best = xch[pl.ds(off, L)]
                bidx = idxs[0]
                for k in range(1, NCHUNK):
                    v = xch[pl.ds(off + k * L, L)]
                    upd = v < best
                    best = jnp.minimum(best, v)
                    bidx = jnp.where(upd, idxs[k], bidx)
                tbuf[pl.ds(rr * PAD, L)] = best
                ibuf[pl.ds(rr * PAD, L)] = bidx
            cols = [
                plsc.load_gather(tbuf, [lane_pad + jnp.int32(c)])
                for c in range(L)
            ]
            rowmin = cols[0]
            for c in range(1, L):
                rowmin = jnp.minimum(rowmin, cols[c])
            wacc = jnp.full((L,), _BIG, jnp.int32)
            for c in range(L):
                icol = plsc.load_gather(ibuf, [lane_pad + jnp.int32(c)])
                cand = jnp.where(cols[c] == rowmin, icol, jnp.int32(_BIG))
                wacc = jnp.minimum(wacc, cand)
            labs = plsc.load_gather(labv, [wacc])
            outv[pl.ds(g * L, L)] = labs
            return carry

        lax.fori_loop(0, CH // L, group_step, 0)
        pltpu.async_copy(
            outv, out_hbm.at[pl.ds(wid * ROWS_W + ch * CH, CH)], osem
        ).wait()


def kernel(x, prototype_labels):
    run = functools.partial(
        pl.kernel,
        mesh=plsc.VectorSubcoreMesh(core_axis_name="c", subcore_axis_name="s"),
        out_type=jax.ShapeDtypeStruct((B,), jnp.float32),
        compiler_params=pltpu.CompilerParams(needs_layout_passes=False),
        scratch_types=[
            pltpu.VMEM((CH * N,), jnp.float32),
            pltpu.VMEM((CH * N,), jnp.float32),
            pltpu.VMEM((N,), jnp.float32),
            pltpu.VMEM((CH,), jnp.float32),
            pltpu.VMEM((L * PAD,), jnp.float32),
            pltpu.VMEM((L * PAD,), jnp.int32),
            pltpu.SemaphoreType.DMA,
            pltpu.SemaphoreType.DMA,
            pltpu.SemaphoreType.DMA,
        ],
    )(_wtac_body)
    return run(x.reshape(B * N), prototype_labels)


# speedup vs baseline: 1.4108x; 1.4108x over previous
import functools

import jax
import jax.numpy as jnp
from jax import lax
from jax.experimental import pallas as pl
from jax.experimental.pallas import tpu as pltpu
from jax.experimental.pallas import tpu_sc as plsc

B = 16384
N = 256
L = 16
NCHUNK = N // L
PAD = L + 1

_info = plsc.get_sparse_core_info()
NC, NS = _info.num_cores, _info.num_subcores
NW = NC * NS
ROWS_W = B // NW
CH = 64
NCH = ROWS_W // CH

_BIG = 2**30


def _wtac_body(x_hbm, lab_hbm, out_hbm, xb0, xb1, labv, outv, tbuf, ibuf,
               sem0, sem1, osem):
    wid = lax.axis_index("s") * NC + lax.axis_index("c")
    base = wid * ROWS_W
    pltpu.sync_copy(lab_hbm, labv)
    bufs = (xb0, xb1)
    sems = (sem0, sem1)
    copies = [None] * NCH
    copies[0] = pltpu.async_copy(
        x_hbm.at[pl.ds(base, CH), :], xb0, sems[0]
    )
    lane = lax.iota(jnp.int32, L)
    idxs = [lane + jnp.int32(k * L) for k in range(NCHUNK)]
    lane_pad = lane * jnp.int32(PAD)

    for ch in range(NCH):
        if ch + 1 < NCH:
            copies[ch + 1] = pltpu.async_copy(
                x_hbm.at[pl.ds(base + (ch + 1) * CH, CH), :],
                bufs[(ch + 1) % 2],
                sems[(ch + 1) % 2],
            )
        copies[ch].wait()
        xch = bufs[ch % 2]

        def group_step(g, carry, xch=xch):
            for rr in range(L):
                row = g * L + rr
                best = xch[row, pl.ds(0, L)]
                bidx = idxs[0]
                for k in range(1, NCHUNK):
                    v = xch[row, pl.ds(k * L, L)]
                    upd = v < best
                    best = jnp.minimum(best, v)
                    bidx = jnp.where(upd, idxs[k], bidx)
                tbuf[pl.ds(rr * PAD, L)] = best
                ibuf[pl.ds(rr * PAD, L)] = bidx
            cols = [
                plsc.load_gather(tbuf, [lane_pad + jnp.int32(c)])
                for c in range(L)
            ]
            rowmin = cols[0]
            for c in range(1, L):
                rowmin = jnp.minimum(rowmin, cols[c])
            wacc = jnp.full((L,), _BIG, jnp.int32)
            for c in range(L):
                icol = plsc.load_gather(ibuf, [lane_pad + jnp.int32(c)])
                cand = jnp.where(cols[c] == rowmin, icol, jnp.int32(_BIG))
                wacc = jnp.minimum(wacc, cand)
            labs = plsc.load_gather(labv, [wacc])
            outv[pl.ds(g * L, L)] = labs
            return carry

        lax.fori_loop(0, CH // L, group_step, 0)
        pltpu.async_copy(
            outv, out_hbm.at[pl.ds(base + ch * CH, CH)], osem
        ).wait()


def kernel(x, prototype_labels):
    run = functools.partial(
        pl.kernel,
        mesh=plsc.VectorSubcoreMesh(core_axis_name="c", subcore_axis_name="s"),
        out_type=jax.ShapeDtypeStruct((B,), jnp.float32),
        compiler_params=pltpu.CompilerParams(needs_layout_passes=False),
        scratch_types=[
            pltpu.VMEM((CH, N), jnp.float32),
            pltpu.VMEM((CH, N), jnp.float32),
            pltpu.VMEM((N,), jnp.float32),
            pltpu.VMEM((CH,), jnp.float32),
            pltpu.VMEM((L * PAD,), jnp.float32),
            pltpu.VMEM((L * PAD,), jnp.int32),
            pltpu.SemaphoreType.DMA,
            pltpu.SemaphoreType.DMA,
            pltpu.SemaphoreType.DMA,
        ],
    )(_wtac_body)
    return run(x, prototype_labels)


# speedup vs baseline: 3.6119x; 2.5601x over previous
import jax
import jax.numpy as jnp
from jax.experimental import pallas as pl

B = 16384
N = 256
BLOCK_ROWS = 1024
NUM_BLOCKS = B // BLOCK_ROWS


def _wtac_block(x_ref, lab_ref, out_ref):
    x = x_ref[...]
    m = jnp.min(x, axis=1, keepdims=True)
    colf = jax.lax.broadcasted_iota(jnp.int32, (BLOCK_ROWS, N), 1).astype(
        jnp.float32
    )
    cand = jnp.where(x == m, colf, jnp.float32(2.0**30))
    win = jnp.min(cand, axis=1, keepdims=True)
    onehot = (cand == win).astype(jnp.float32)
    lab = lab_ref[...]
    y = jax.lax.dot_general(
        lab, onehot, (((1,), (1,)), ((), ())),
        preferred_element_type=jnp.float32,
    )
    out_ref[...] = y.reshape(BLOCK_ROWS)


def kernel(x, prototype_labels):
    lab2d = prototype_labels.reshape(1, N)
    out = pl.pallas_call(
        _wtac_block,
        grid=(NUM_BLOCKS,),
        in_specs=[
            pl.BlockSpec((BLOCK_ROWS, N), lambda i: (i, 0)),
            pl.BlockSpec((1, N), lambda i: (0, 0)),
        ],
        out_specs=pl.BlockSpec((BLOCK_ROWS,), lambda i: (i,)),
        out_shape=jax.ShapeDtypeStruct((B,), jnp.float32),
    )(x, lab2d)
    return out


# speedup vs baseline: 5.0159x; 1.3887x over previous
import jax
import jax.numpy as jnp
from jax.experimental import pallas as pl

B = 16384
N = 256
BLOCK_ROWS = 2048
NUM_BLOCKS = B // BLOCK_ROWS


def _wtac_block(x_ref, lab_ref, out_ref):
    x = x_ref[...]
    m = jnp.min(x, axis=1, keepdims=True)
    colf = jax.lax.broadcasted_iota(jnp.int32, (BLOCK_ROWS, N), 1).astype(
        jnp.float32
    )
    cand = jnp.where(x == m, colf, jnp.float32(2.0**30))
    win = jnp.min(cand, axis=1, keepdims=True)
    onehot = (cand == win).astype(jnp.float32)
    lab = lab_ref[...]
    y = jax.lax.dot_general(
        lab, onehot, (((1,), (1,)), ((), ())),
        preferred_element_type=jnp.float32,
    )
    out_ref[...] = y.reshape(BLOCK_ROWS)


def kernel(x, prototype_labels):
    lab2d = prototype_labels.reshape(1, N)
    out = pl.pallas_call(
        _wtac_block,
        grid=(NUM_BLOCKS,),
        in_specs=[
            pl.BlockSpec((BLOCK_ROWS, N), lambda i: (i, 0)),
            pl.BlockSpec((1, N), lambda i: (0, 0)),
        ],
        out_specs=pl.BlockSpec((BLOCK_ROWS,), lambda i: (i,)),
        out_shape=jax.ShapeDtypeStruct((B,), jnp.float32),
    )(x, lab2d)
    return out


# speedup vs baseline: 6.0847x; 1.2131x over previous
import jax
import jax.numpy as jnp
from jax.experimental import pallas as pl

B = 16384
N = 256
BLOCK_ROWS = 4096
NUM_BLOCKS = B // BLOCK_ROWS


def _wtac_block(x_ref, lab_ref, out_ref):
    x = x_ref[...]
    m = jnp.min(x, axis=1, keepdims=True)
    colf = jax.lax.broadcasted_iota(jnp.int32, (BLOCK_ROWS, N), 1).astype(
        jnp.float32
    )
    cand = jnp.where(x == m, colf, jnp.float32(2.0**30))
    win = jnp.min(cand, axis=1, keepdims=True)
    onehot = (cand == win).astype(jnp.float32)
    lab = lab_ref[...]
    y = jax.lax.dot_general(
        lab, onehot, (((1,), (1,)), ((), ())),
        preferred_element_type=jnp.float32,
    )
    out_ref[...] = y.reshape(BLOCK_ROWS)


def kernel(x, prototype_labels):
    lab2d = prototype_labels.reshape(1, N)
    out = pl.pallas_call(
        _wtac_block,
        grid=(NUM_BLOCKS,),
        in_specs=[
            pl.BlockSpec((BLOCK_ROWS, N), lambda i: (i, 0)),
            pl.BlockSpec((1, N), lambda i: (0, 0)),
        ],
        out_specs=pl.BlockSpec((BLOCK_ROWS,), lambda i: (i,)),
        out_shape=jax.ShapeDtypeStruct((B,), jnp.float32),
    )(x, lab2d)
    return out


# speedup vs baseline: 6.1407x; 1.0092x over previous
import jax
import jax.numpy as jnp
from jax.experimental import pallas as pl

B = 16384
N = 256
BLOCK_ROWS = 8192
NUM_BLOCKS = B // BLOCK_ROWS


def _wtac_block(x_ref, lab_ref, out_ref):
    x = x_ref[...]
    m = jnp.min(x, axis=1, keepdims=True)
    colf = jax.lax.broadcasted_iota(jnp.int32, (BLOCK_ROWS, N), 1).astype(
        jnp.float32
    )
    cand = jnp.where(x == m, colf, jnp.float32(2.0**30))
    win = jnp.min(cand, axis=1, keepdims=True)
    onehot = (cand == win).astype(jnp.float32)
    lab = lab_ref[...]
    y = jax.lax.dot_general(
        lab, onehot, (((1,), (1,)), ((), ())),
        preferred_element_type=jnp.float32,
    )
    out_ref[...] = y.reshape(BLOCK_ROWS)


def kernel(x, prototype_labels):
    lab2d = prototype_labels.reshape(1, N)
    out = pl.pallas_call(
        _wtac_block,
        grid=(NUM_BLOCKS,),
        in_specs=[
            pl.BlockSpec((BLOCK_ROWS, N), lambda i: (i, 0)),
            pl.BlockSpec((1, N), lambda i: (0, 0)),
        ],
        out_specs=pl.BlockSpec((BLOCK_ROWS,), lambda i: (i,)),
        out_shape=jax.ShapeDtypeStruct((B,), jnp.float32),
    )(x, lab2d)
    return out
